# baseline (speedup 1.0000x reference)
import jax
import jax.numpy as jnp
from jax.experimental import pallas as pl

_B, _C = 4096, 1000
_BR = 1024
_GRID = _B // _BR


def _body(x_ref, yt_ref, perm_ref, ytfull_ref, lam_ref, out_ref):
    i = pl.program_id(0)
    x = x_ref[:, :]
    m = jnp.max(x, axis=1, keepdims=True)
    s = jnp.sum(jnp.exp(x - m), axis=1, keepdims=True)
    lse = m + jnp.log(s)

    col = jax.lax.broadcasted_iota(jnp.int32, (_BR, _C), 1)
    p0 = jnp.sum(jnp.where(col == yt_ref[:, :], x, 0.0), axis=1, keepdims=True)

    rows = jax.lax.broadcasted_iota(jnp.int32, (_BR, _B), 1)
    labels1 = jnp.sum(jnp.where(rows == perm_ref[:, :], ytfull_ref[:, :], 0),
                      axis=1, keepdims=True)
    p1 = jnp.sum(jnp.where(col == labels1, x, 0.0), axis=1, keepdims=True)

    lam = lam_ref[:, :]
    part = (jnp.sum(lse, axis=0, keepdims=True)
            - lam * jnp.sum(p0, axis=0, keepdims=True)
            - (1.0 - lam) * jnp.sum(p1, axis=0, keepdims=True))

    @pl.when(i == 0)
    def _init():
        out_ref[:, :] = jnp.zeros_like(out_ref)

    out_ref[:, :] += part

    @pl.when(i == _GRID - 1)
    def _fin():
        out_ref[:, :] = out_ref[:, :] * (1.0 / _B)


def kernel(y_pred, y_true, perm_index, lam):
    lam_arr = jnp.asarray(lam, jnp.float32).reshape(1, 1)
    out = pl.pallas_call(
        _body,
        grid=(_GRID,),
        in_specs=[
            pl.BlockSpec((_BR, _C), lambda i: (i, 0)),
            pl.BlockSpec((_BR, 1), lambda i: (i, 0)),
            pl.BlockSpec((_BR, 1), lambda i: (i, 0)),
            pl.BlockSpec((1, _B), lambda i: (0, 0)),
            pl.BlockSpec((1, 1), lambda i: (0, 0)),
        ],
        out_specs=pl.BlockSpec((1, 1), lambda i: (0, 0)),
        out_shape=jax.ShapeDtypeStruct((1, 1), jnp.float32),
    )(y_pred, y_true.reshape(_B, 1), perm_index.reshape(_B, 1),
      y_true.reshape(1, _B), lam_arr)
    return out.reshape(())


# speedup vs baseline: 1.1086x; 1.1086x over previous
import jax
import jax.numpy as jnp
from jax.experimental import pallas as pl

_B, _C = 4096, 1000
_BR = 1024
_GRID = _B // _BR


def _body(x_ref, yt_ref, perm_ref, ytsq_ref, lam_ref, out_ref):
    i = pl.program_id(0)
    x = x_ref[:, :]
    m = jnp.max(x, axis=1, keepdims=True)
    s = jnp.sum(jnp.exp(x - m), axis=1, keepdims=True)
    lse = m + jnp.log(s)

    perm_blk = perm_ref[:, :]
    biota = jax.lax.broadcasted_iota(jnp.int32, (_BR, 64), 1)
    u = jnp.where(biota == (perm_blk & 63), 1.0, 0.0).astype(jnp.float32)
    v = jnp.where(biota == (perm_blk >> 6), 1.0, 0.0).astype(jnp.float32)
    t = jax.lax.dot_general(u, ytsq_ref[:, :],
                            (((1,), (1,)), ((), ())),
                            preferred_element_type=jnp.float32)
    labels1 = jnp.sum(v * t, axis=1, keepdims=True).astype(jnp.int32)

    col = jax.lax.broadcasted_iota(jnp.int32, (_BR, _C), 1)
    p0 = jnp.sum(jnp.where(col == yt_ref[:, :], x, 0.0), axis=1, keepdims=True)
    p1 = jnp.sum(jnp.where(col == labels1, x, 0.0), axis=1, keepdims=True)

    lam = lam_ref[:, :]
    part = (jnp.sum(lse, axis=0, keepdims=True)
            - lam * jnp.sum(p0, axis=0, keepdims=True)
            - (1.0 - lam) * jnp.sum(p1, axis=0, keepdims=True))

    @pl.when(i == 0)
    def _init():
        out_ref[:, :] = jnp.zeros_like(out_ref)

    out_ref[:, :] += part

    @pl.when(i == _GRID - 1)
    def _fin():
        out_ref[:, :] = out_ref[:, :] * (1.0 / _B)


def kernel(y_pred, y_true, perm_index, lam):
    lam_arr = jnp.asarray(lam, jnp.float32).reshape(1, 1)
    ytsq = y_true.astype(jnp.float32).reshape(64, 64)
    out = pl.pallas_call(
        _body,
        grid=(_GRID,),
        in_specs=[
            pl.BlockSpec((_BR, _C), lambda i: (i, 0)),
            pl.BlockSpec((_BR, 1), lambda i: (i, 0)),
            pl.BlockSpec((_BR, 1), lambda i: (i, 0)),
            pl.BlockSpec((64, 64), lambda i: (0, 0)),
            pl.BlockSpec((1, 1), lambda i: (0, 0)),
        ],
        out_specs=pl.BlockSpec((1, 1), lambda i: (0, 0)),
        out_shape=jax.ShapeDtypeStruct((1, 1), jnp.float32),
    )(y_pred, y_true.reshape(_B, 1), perm_index.reshape(_B, 1), ytsq, lam_arr)
    return out.reshape(())
